# baseline (speedup 1.0000x reference)
import functools

import jax
import jax.numpy as jnp
from jax import lax
from jax.experimental import pallas as pl
from jax.experimental.pallas import tpu as pltpu
from jax.experimental.pallas import tpu_sc as plsc

N = 25000
D = 128
H = 64
E = 400000

NS = 16
ECHUNK = 96
NCHUNK = 272
NBLK = 34
BCH = NCHUNK // NBLK
NBUF = 4
DEPTH = NBUF - 1
EPT = NCHUNK * ECHUNK
E_PAD = NS * EPT
RPT = 1600
N_PAD = NS * RPT

RB = 5000


def _dot_t(a, b):
    return lax.dot_general(a, b, (((1,), (1,)), ((), ())),
                           preferred_element_type=jnp.float32)


def _proj_body(x0, x1, wr0, wc0, br0, wr1, wc1, br1, y0, y1, c0, c1):
    xv0 = x0[...]
    xv1 = x1[...]
    y0[...] = _dot_t(xv0, wr0[...])
    c0[...] = _dot_t(xv0, wc0[...]) + br0[...]
    y1[...] = _dot_t(xv1, wr1[...])
    c1[...] = _dot_t(xv1, wc1[...]) + br1[...]


_proj_call = pl.pallas_call(
    _proj_body,
    grid=(N // RB,),
    in_specs=[
        pl.BlockSpec((RB, D), lambda i: (i, 0)),
        pl.BlockSpec((RB, D), lambda i: (i, 0)),
        pl.BlockSpec((H, D), lambda i: (0, 0)),
        pl.BlockSpec((H, D), lambda i: (0, 0)),
        pl.BlockSpec((1, H), lambda i: (0, 0)),
        pl.BlockSpec((H, D), lambda i: (0, 0)),
        pl.BlockSpec((H, D), lambda i: (0, 0)),
        pl.BlockSpec((1, H), lambda i: (0, 0)),
    ],
    out_specs=[pl.BlockSpec((RB, H), lambda i: (i, 0))] * 4,
    out_shape=[jax.ShapeDtypeStruct((N, H), jnp.float32)] * 4,
)


def _sc_body(y0, y1, src0, dst0, src1, dst1, agg0, agg1,
             idx_src, idx_dst, rows0, rows1, rows2, rows3,
             gsem0, gsem1, gsem2, gsem3, ssem0, ssem1, ssem2, ssem3, acc):
    cid = lax.axis_index("c")
    sid = lax.axis_index("s")
    rbase = sid * RPT

    def zrow(i, c):
        for j in range(H // 16):
            rows0[i, pl.ds(j * 16, 16)] = jnp.zeros((16,), jnp.float32)
        return c
    lax.fori_loop(0, ECHUNK, zrow, 0, unroll=False)

    def zcopy(k, c):
        pltpu.sync_copy(rows0, acc.at[pl.ds(rbase + k * ECHUNK, ECHUNK)])
        return c
    lax.fori_loop(0, RPT // ECHUNK, zcopy, 0, unroll=False)
    pltpu.sync_copy(rows0.at[pl.ds(0, RPT % ECHUNK)],
                    acc.at[pl.ds(rbase + (RPT // ECHUNK) * ECHUNK,
                                 RPT % ECHUNK)])
    plsc.subcore_barrier()

    rows = (rows0, rows1, rows2, rows3)
    gsems = (gsem0, gsem1, gsem2, gsem3)
    ssems = (ssem0, ssem1, ssem2, ssem3)

    def run(y, src, dst, agg):
        def blk(b, c):
            base = sid * NCHUNK + b * BCH
            pltpu.sync_copy(src.at[pl.ds(base, BCH)], idx_src)
            pltpu.sync_copy(dst.at[pl.ds(base, BCH)], idx_dst)

            for k in range(DEPTH):
                pltpu.async_copy(y.at[idx_src.at[k]], rows[k], gsems[k])
            for k in range(BCH):
                p = k % NBUF
                pltpu.make_async_copy(y.at[idx_src.at[k]],
                                      rows[p], gsems[p]).wait()
                if k >= 1:
                    q = (k - 1) % NBUF
                    pltpu.make_async_copy(rows[q], acc.at[idx_dst.at[k - 1]],
                                          ssems[q]).wait()
                if k + DEPTH < BCH:
                    r = (k + DEPTH) % NBUF
                    pltpu.async_copy(y.at[idx_src.at[k + DEPTH]], rows[r],
                                     gsems[r])
                pltpu.async_copy(rows[p], acc.at[idx_dst.at[k]], ssems[p],
                                 add=True)
            p = (BCH - 1) % NBUF
            pltpu.make_async_copy(rows[p], acc.at[idx_dst.at[BCH - 1]],
                                  ssems[p]).wait()
            return c
        lax.fori_loop(0, NBLK, blk, 0, unroll=False)

        plsc.subcore_barrier()
        pltpu.sync_copy(acc.at[pl.ds(rbase, RPT)], agg.at[pl.ds(rbase, RPT)])

    @pl.when(cid == 0)
    def _():
        run(y0, src0, dst0, agg0)

    @pl.when(cid == 1)
    def _():
        run(y1, src1, dst1, agg1)


_sc_call = pl.kernel(
    _sc_body,
    out_type=(jax.ShapeDtypeStruct((N_PAD, H), jnp.float32),
              jax.ShapeDtypeStruct((N_PAD, H), jnp.float32)),
    mesh=plsc.VectorSubcoreMesh(core_axis_name="c", subcore_axis_name="s"),
    compiler_params=pltpu.CompilerParams(use_tc_tiling_on_sc=False),
    scratch_types=[
        pltpu.VMEM((BCH, ECHUNK), jnp.int32),
        pltpu.VMEM((BCH, ECHUNK), jnp.int32),
        pltpu.VMEM((ECHUNK, H), jnp.float32),
        pltpu.VMEM((ECHUNK, H), jnp.float32),
        pltpu.VMEM((ECHUNK, H), jnp.float32),
        pltpu.VMEM((ECHUNK, H), jnp.float32),
        pltpu.SemaphoreType.DMA,
        pltpu.SemaphoreType.DMA,
        pltpu.SemaphoreType.DMA,
        pltpu.SemaphoreType.DMA,
        pltpu.SemaphoreType.DMA,
        pltpu.SemaphoreType.DMA,
        pltpu.SemaphoreType.DMA,
        pltpu.SemaphoreType.DMA,
        pltpu.VMEM_SHARED((N_PAD, H), jnp.float32),
    ],
)


def _out_body(a0, c0, a1, c1, wfc, bfc, o):
    h0 = jnp.maximum(a0[...] + c0[...], 0.0)
    h1 = jnp.maximum(a1[...] + c1[...], 0.0)
    w = wfc[...]
    s = h0 * w[:, :H] + h1 * w[:, H:]
    o[...] = jnp.sum(s, axis=1, keepdims=True) + bfc[0, 0]


_out_call = pl.pallas_call(
    _out_body,
    grid=(N // RB,),
    in_specs=[
        pl.BlockSpec((RB, H), lambda i: (i, 0)),
        pl.BlockSpec((RB, H), lambda i: (i, 0)),
        pl.BlockSpec((RB, H), lambda i: (i, 0)),
        pl.BlockSpec((RB, H), lambda i: (i, 0)),
        pl.BlockSpec((1, 2 * H), lambda i: (0, 0)),
        pl.BlockSpec((1, 1), lambda i: (0, 0)),
    ],
    out_specs=pl.BlockSpec((RB, 1), lambda i: (i, 0)),
    out_shape=jax.ShapeDtypeStruct((N, 1), jnp.float32),
)


def _prep_edges(ei):
    src = ei[0].astype(jnp.int32)
    dst = ei[1].astype(jnp.int32)
    pad = E_PAD - E
    src = jnp.concatenate([src, jnp.zeros((pad,), jnp.int32)])
    dump = N + (jnp.arange(pad, dtype=jnp.int32) % (N_PAD - N))
    dst = jnp.concatenate([dst, dump])
    return (src.reshape(E_PAD // ECHUNK, ECHUNK),
            dst.reshape(E_PAD // ECHUNK, ECHUNK))


def kernel(x_v0, x_v1, edge_index_v0v1, edge_index_v1v0,
           W_rel0, b_rel0, W_root0, W_rel1, b_rel1, W_root1, W_fc, b_fc):
    y0, y1, c0, c1 = _proj_call(
        x_v0, x_v1,
        W_rel0, W_root0, b_rel0.reshape(1, H),
        W_rel1, W_root1, b_rel1.reshape(1, H))
    s0, d0 = _prep_edges(edge_index_v0v1)
    s1, d1 = _prep_edges(edge_index_v1v0)
    agg0, agg1 = _sc_call(y0, y1, s0, d0, s1, d1)
    out = _out_call(agg0, c0, agg1, c1, W_fc, b_fc.reshape(1, 1))
    return out


# speedup vs baseline: 1.4086x; 1.4086x over previous
import functools

import jax
import jax.numpy as jnp
from jax import lax
from jax.experimental import pallas as pl
from jax.experimental.pallas import tpu as pltpu
from jax.experimental.pallas import tpu_sc as plsc

N = 25000
D = 128
H = 64
HH = H // 2
E = 400000

NS = 16
ECHUNK = 96
NCHUNK = 272
NBLK = 34
BCH = NCHUNK // NBLK
NBUF = 4
DEPTH = NBUF - 1
EPT = NCHUNK * ECHUNK
E_PAD = NS * EPT
RPT = 1600
N_PAD = NS * RPT

RB = 5000


def _dot_t(a, b):
    return lax.dot_general(a, b, (((1,), (1,)), ((), ())),
                           preferred_element_type=jnp.float32)


def _proj_body(x0, x1, wr0, wc0, br0, wr1, wc1, br1, y0, y1, c0, c1):
    xv0 = x0[...]
    xv1 = x1[...]
    y0[...] = _dot_t(xv0, wr0[...])
    c0[...] = _dot_t(xv0, wc0[...]) + br0[...]
    y1[...] = _dot_t(xv1, wr1[...])
    c1[...] = _dot_t(xv1, wc1[...]) + br1[...]


_proj_call = pl.pallas_call(
    _proj_body,
    grid=(N // RB,),
    in_specs=[
        pl.BlockSpec((RB, D), lambda i: (i, 0)),
        pl.BlockSpec((RB, D), lambda i: (i, 0)),
        pl.BlockSpec((H, D), lambda i: (0, 0)),
        pl.BlockSpec((H, D), lambda i: (0, 0)),
        pl.BlockSpec((1, H), lambda i: (0, 0)),
        pl.BlockSpec((H, D), lambda i: (0, 0)),
        pl.BlockSpec((H, D), lambda i: (0, 0)),
        pl.BlockSpec((1, H), lambda i: (0, 0)),
    ],
    out_specs=[pl.BlockSpec((RB, H), lambda i: (i, 0))] * 4,
    out_shape=[jax.ShapeDtypeStruct((N, H), jnp.float32)] * 4,
)


def _sc_body(y0, y1, src0, dst0, src1, dst1, agg00, agg01, agg10, agg11,
             idx_src, idx_dst, rows0, rows1, rows2, rows3,
             gsem0, gsem1, gsem2, gsem3, ssem0, ssem1, ssem2, ssem3,
             table, acc):
    cid = lax.axis_index("c")
    sid = lax.axis_index("s")
    rbase = sid * RPT

    rows = (rows0, rows1, rows2, rows3)
    gsems = (gsem0, gsem1, gsem2, gsem3)
    ssems = (ssem0, ssem1, ssem2, ssem3)

    def zrow(i, c):
        for j in range(HH // 16):
            rows0[i, pl.ds(j * 16, 16)] = jnp.zeros((16,), jnp.float32)
        return c
    lax.fori_loop(0, ECHUNK, zrow, 0, unroll=False)

    def run_pass(y, src, dst, agg, h):
        pltpu.sync_copy(y.at[pl.ds(rbase, RPT), pl.ds(h * HH, HH)],
                        table.at[pl.ds(rbase, RPT)])

        def zcopy(k, c):
            pltpu.sync_copy(rows0, acc.at[pl.ds(rbase + k * ECHUNK, ECHUNK)])
            return c
        lax.fori_loop(0, RPT // ECHUNK, zcopy, 0, unroll=False)
        pltpu.sync_copy(rows0.at[pl.ds(0, RPT % ECHUNK)],
                        acc.at[pl.ds(rbase + (RPT // ECHUNK) * ECHUNK,
                                     RPT % ECHUNK)])
        plsc.subcore_barrier()

        def blk(b, c):
            base = sid * NCHUNK + b * BCH
            pltpu.sync_copy(src.at[pl.ds(base, BCH)], idx_src)
            pltpu.sync_copy(dst.at[pl.ds(base, BCH)], idx_dst)

            for k in range(DEPTH):
                pltpu.async_copy(table.at[idx_src.at[k]], rows[k], gsems[k])
            for k in range(BCH):
                p = k % NBUF
                pltpu.make_async_copy(table.at[idx_src.at[k]],
                                      rows[p], gsems[p]).wait()
                if k >= 1:
                    q = (k - 1) % NBUF
                    pltpu.make_async_copy(rows[q], acc.at[idx_dst.at[k - 1]],
                                          ssems[q]).wait()
                if k + DEPTH < BCH:
                    r = (k + DEPTH) % NBUF
                    pltpu.async_copy(table.at[idx_src.at[k + DEPTH]],
                                     rows[r], gsems[r])
                pltpu.async_copy(rows[p], acc.at[idx_dst.at[k]], ssems[p],
                                 add=True)
            p = (BCH - 1) % NBUF
            pltpu.make_async_copy(rows[p], acc.at[idx_dst.at[BCH - 1]],
                                  ssems[p]).wait()
            return c
        lax.fori_loop(0, NBLK, blk, 0, unroll=False)

        plsc.subcore_barrier()
        pltpu.sync_copy(acc.at[pl.ds(rbase, RPT)], agg.at[pl.ds(rbase, RPT)])
        plsc.subcore_barrier()

    @pl.when(cid == 0)
    def _():
        run_pass(y0, src0, dst0, agg00, 0)
        run_pass(y0, src0, dst0, agg01, 1)

    @pl.when(cid == 1)
    def _():
        run_pass(y1, src1, dst1, agg10, 0)
        run_pass(y1, src1, dst1, agg11, 1)


_sc_call = pl.kernel(
    _sc_body,
    out_type=tuple(jax.ShapeDtypeStruct((N_PAD, HH), jnp.float32)
                   for _ in range(4)),
    mesh=plsc.VectorSubcoreMesh(core_axis_name="c", subcore_axis_name="s"),
    compiler_params=pltpu.CompilerParams(use_tc_tiling_on_sc=False),
    scratch_types=[
        pltpu.VMEM((BCH, ECHUNK), jnp.int32),
        pltpu.VMEM((BCH, ECHUNK), jnp.int32),
        pltpu.VMEM((ECHUNK, HH), jnp.float32),
        pltpu.VMEM((ECHUNK, HH), jnp.float32),
        pltpu.VMEM((ECHUNK, HH), jnp.float32),
        pltpu.VMEM((ECHUNK, HH), jnp.float32),
        pltpu.SemaphoreType.DMA,
        pltpu.SemaphoreType.DMA,
        pltpu.SemaphoreType.DMA,
        pltpu.SemaphoreType.DMA,
        pltpu.SemaphoreType.DMA,
        pltpu.SemaphoreType.DMA,
        pltpu.SemaphoreType.DMA,
        pltpu.SemaphoreType.DMA,
        pltpu.VMEM_SHARED((N_PAD, HH), jnp.float32),
        pltpu.VMEM_SHARED((N_PAD, HH), jnp.float32),
    ],
)


def _out_body(a00, a01, a10, a11, c0, c1, wfc, bfc, o):
    cv0 = c0[...]
    cv1 = c1[...]
    w = wfc[...]
    h0a = jnp.maximum(a00[...] + cv0[:, :HH], 0.0)
    h0b = jnp.maximum(a01[...] + cv0[:, HH:], 0.0)
    h1a = jnp.maximum(a10[...] + cv1[:, :HH], 0.0)
    h1b = jnp.maximum(a11[...] + cv1[:, HH:], 0.0)
    s = (h0a * w[:, 0:HH] + h0b * w[:, HH:H]
         + h1a * w[:, H:H + HH] + h1b * w[:, H + HH:])
    o[...] = jnp.sum(s, axis=1, keepdims=True) + bfc[0, 0]


_out_call = pl.pallas_call(
    _out_body,
    grid=(N // RB,),
    in_specs=[
        pl.BlockSpec((RB, HH), lambda i: (i, 0)),
        pl.BlockSpec((RB, HH), lambda i: (i, 0)),
        pl.BlockSpec((RB, HH), lambda i: (i, 0)),
        pl.BlockSpec((RB, HH), lambda i: (i, 0)),
        pl.BlockSpec((RB, H), lambda i: (i, 0)),
        pl.BlockSpec((RB, H), lambda i: (i, 0)),
        pl.BlockSpec((1, 2 * H), lambda i: (0, 0)),
        pl.BlockSpec((1, 1), lambda i: (0, 0)),
    ],
    out_specs=pl.BlockSpec((RB, 1), lambda i: (i, 0)),
    out_shape=jax.ShapeDtypeStruct((N, 1), jnp.float32),
)


def _prep_edges(ei):
    src = ei[0].astype(jnp.int32)
    dst = ei[1].astype(jnp.int32)
    pad = E_PAD - E
    src = jnp.concatenate([src, jnp.arange(pad, dtype=jnp.int32) % N])
    dump = N + (jnp.arange(pad, dtype=jnp.int32) % (N_PAD - N))
    dst = jnp.concatenate([dst, dump])
    return (src.reshape(E_PAD // ECHUNK, ECHUNK),
            dst.reshape(E_PAD // ECHUNK, ECHUNK))


def kernel(x_v0, x_v1, edge_index_v0v1, edge_index_v1v0,
           W_rel0, b_rel0, W_root0, W_rel1, b_rel1, W_root1, W_fc, b_fc):
    y0, y1, c0, c1 = _proj_call(
        x_v0, x_v1,
        W_rel0, W_root0, b_rel0.reshape(1, H),
        W_rel1, W_root1, b_rel1.reshape(1, H))
    y0p = jnp.pad(y0, ((0, N_PAD - N), (0, 0)))
    y1p = jnp.pad(y1, ((0, N_PAD - N), (0, 0)))
    s0, d0 = _prep_edges(edge_index_v0v1)
    s1, d1 = _prep_edges(edge_index_v1v0)
    agg00, agg01, agg10, agg11 = _sc_call(y0p, y1p, s0, d0, s1, d1)
    out = _out_call(agg00, agg01, agg10, agg11, c0, c1,
                    W_fc, b_fc.reshape(1, 1))
    return out


# speedup vs baseline: 1.4863x; 1.0551x over previous
import functools

import jax
import jax.numpy as jnp
from jax import lax
from jax.experimental import pallas as pl
from jax.experimental.pallas import tpu as pltpu
from jax.experimental.pallas import tpu_sc as plsc

N = 25000
D = 128
H = 64
HH = H // 2
E = 400000

NS = 16
ECHUNK = 128
NCHUNK = 200
NBLK = 20
BCH = NCHUNK // NBLK
NBUF = 3
DEPTH = NBUF - 1
EPT = NCHUNK * ECHUNK
E_PAD = NS * EPT
RPT = 1600
N_PAD = NS * RPT

RB = 5000


def _dot_t(a, b):
    return lax.dot_general(a, b, (((1,), (1,)), ((), ())),
                           preferred_element_type=jnp.float32)


def _proj_body(x0, x1, wr0, wc0, br0, wr1, wc1, br1, y0, y1, c0, c1):
    xv0 = x0[...]
    xv1 = x1[...]
    y0[...] = _dot_t(xv0, wr0[...])
    c0[...] = _dot_t(xv0, wc0[...]) + br0[...]
    y1[...] = _dot_t(xv1, wr1[...])
    c1[...] = _dot_t(xv1, wc1[...]) + br1[...]


_proj_call = pl.pallas_call(
    _proj_body,
    grid=(N // RB,),
    in_specs=[
        pl.BlockSpec((RB, D), lambda i: (i, 0)),
        pl.BlockSpec((RB, D), lambda i: (i, 0)),
        pl.BlockSpec((H, D), lambda i: (0, 0)),
        pl.BlockSpec((H, D), lambda i: (0, 0)),
        pl.BlockSpec((1, H), lambda i: (0, 0)),
        pl.BlockSpec((H, D), lambda i: (0, 0)),
        pl.BlockSpec((H, D), lambda i: (0, 0)),
        pl.BlockSpec((1, H), lambda i: (0, 0)),
    ],
    out_specs=[pl.BlockSpec((RB, H), lambda i: (i, 0))] * 4,
    out_shape=[jax.ShapeDtypeStruct((N, H), jnp.float32)] * 4,
)


def _sc_body(y0, y1, src0, dst0, src1, dst1, agg00, agg01, agg10, agg11,
             idx_src, idx_dst, rows0, rows1, rows2,
             gsem0, gsem1, gsem2, ssem0, ssem1, ssem2,
             table, acc):
    cid = lax.axis_index("c")
    sid = lax.axis_index("s")
    rbase = sid * RPT

    rows = (rows0, rows1, rows2)
    gsems = (gsem0, gsem1, gsem2)
    ssems = (ssem0, ssem1, ssem2)

    def run_pass(y, src, dst, agg, h):
        pltpu.sync_copy(y.at[pl.ds(rbase, RPT), pl.ds(h * HH, HH)],
                        table.at[pl.ds(rbase, RPT)])

        def zrow(i, c):
            for j in range(HH // 16):
                rows0[i, pl.ds(j * 16, 16)] = jnp.zeros((16,), jnp.float32)
            return c
        lax.fori_loop(0, ECHUNK, zrow, 0, unroll=False)

        def zcopy(k, c):
            pltpu.sync_copy(rows0, acc.at[pl.ds(rbase + k * ECHUNK, ECHUNK)])
            return c
        lax.fori_loop(0, RPT // ECHUNK, zcopy, 0, unroll=False)
        pltpu.sync_copy(rows0.at[pl.ds(0, RPT % ECHUNK)],
                        acc.at[pl.ds(rbase + (RPT // ECHUNK) * ECHUNK,
                                     RPT % ECHUNK)])
        plsc.subcore_barrier()

        def blk(b, c):
            base = sid * NCHUNK + b * BCH
            pltpu.sync_copy(src.at[pl.ds(base, BCH)], idx_src)
            pltpu.sync_copy(dst.at[pl.ds(base, BCH)], idx_dst)

            for k in range(DEPTH):
                pltpu.async_copy(table.at[idx_src.at[k]], rows[k], gsems[k])
            for k in range(BCH):
                p = k % NBUF
                pltpu.make_async_copy(table.at[idx_src.at[k]],
                                      rows[p], gsems[p]).wait()
                if k >= 1:
                    q = (k - 1) % NBUF
                    pltpu.make_async_copy(rows[q], acc.at[idx_dst.at[k - 1]],
                                          ssems[q]).wait()
                if k + DEPTH < BCH:
                    r = (k + DEPTH) % NBUF
                    pltpu.async_copy(table.at[idx_src.at[k + DEPTH]],
                                     rows[r], gsems[r])
                pltpu.async_copy(rows[p], acc.at[idx_dst.at[k]], ssems[p],
                                 add=True)
            p = (BCH - 1) % NBUF
            pltpu.make_async_copy(rows[p], acc.at[idx_dst.at[BCH - 1]],
                                  ssems[p]).wait()
            return c
        lax.fori_loop(0, NBLK, blk, 0, unroll=False)

        plsc.subcore_barrier()
        pltpu.sync_copy(acc.at[pl.ds(rbase, RPT)], agg.at[pl.ds(rbase, RPT)])
        plsc.subcore_barrier()

    @pl.when(cid == 0)
    def _():
        run_pass(y0, src0, dst0, agg00, 0)
        run_pass(y0, src0, dst0, agg01, 1)

    @pl.when(cid == 1)
    def _():
        run_pass(y1, src1, dst1, agg10, 0)
        run_pass(y1, src1, dst1, agg11, 1)


_sc_call = pl.kernel(
    _sc_body,
    out_type=tuple(jax.ShapeDtypeStruct((N_PAD, HH), jnp.float32)
                   for _ in range(4)),
    mesh=plsc.VectorSubcoreMesh(core_axis_name="c", subcore_axis_name="s"),
    compiler_params=pltpu.CompilerParams(use_tc_tiling_on_sc=False),
    scratch_types=[
        pltpu.VMEM((BCH, ECHUNK), jnp.int32),
        pltpu.VMEM((BCH, ECHUNK), jnp.int32),
        pltpu.VMEM((ECHUNK, HH), jnp.float32),
        pltpu.VMEM((ECHUNK, HH), jnp.float32),
        pltpu.VMEM((ECHUNK, HH), jnp.float32),
        pltpu.SemaphoreType.DMA,
        pltpu.SemaphoreType.DMA,
        pltpu.SemaphoreType.DMA,
        pltpu.SemaphoreType.DMA,
        pltpu.SemaphoreType.DMA,
        pltpu.SemaphoreType.DMA,
        pltpu.VMEM_SHARED((N_PAD, HH), jnp.float32),
        pltpu.VMEM_SHARED((N_PAD, HH), jnp.float32),
    ],
)


def _out_body(a00, a01, a10, a11, c0, c1, wfc, bfc, o):
    cv0 = c0[...]
    cv1 = c1[...]
    w = wfc[...]
    h0a = jnp.maximum(a00[...] + cv0[:, :HH], 0.0)
    h0b = jnp.maximum(a01[...] + cv0[:, HH:], 0.0)
    h1a = jnp.maximum(a10[...] + cv1[:, :HH], 0.0)
    h1b = jnp.maximum(a11[...] + cv1[:, HH:], 0.0)
    s = (h0a * w[:, 0:HH] + h0b * w[:, HH:H]
         + h1a * w[:, H:H + HH] + h1b * w[:, H + HH:])
    o[...] = jnp.sum(s, axis=1, keepdims=True) + bfc[0, 0]


_out_call = pl.pallas_call(
    _out_body,
    grid=(N // RB,),
    in_specs=[
        pl.BlockSpec((RB, HH), lambda i: (i, 0)),
        pl.BlockSpec((RB, HH), lambda i: (i, 0)),
        pl.BlockSpec((RB, HH), lambda i: (i, 0)),
        pl.BlockSpec((RB, HH), lambda i: (i, 0)),
        pl.BlockSpec((RB, H), lambda i: (i, 0)),
        pl.BlockSpec((RB, H), lambda i: (i, 0)),
        pl.BlockSpec((1, 2 * H), lambda i: (0, 0)),
        pl.BlockSpec((1, 1), lambda i: (0, 0)),
    ],
    out_specs=pl.BlockSpec((RB, 1), lambda i: (i, 0)),
    out_shape=jax.ShapeDtypeStruct((N, 1), jnp.float32),
)


def _prep_edges(ei):
    src = ei[0].astype(jnp.int32)
    dst = ei[1].astype(jnp.int32)
    pad = E_PAD - E
    src = jnp.concatenate([src, jnp.arange(pad, dtype=jnp.int32) % N])
    dump = N + (jnp.arange(pad, dtype=jnp.int32) % (N_PAD - N))
    dst = jnp.concatenate([dst, dump])
    return (src.reshape(E_PAD // ECHUNK, ECHUNK),
            dst.reshape(E_PAD // ECHUNK, ECHUNK))


def kernel(x_v0, x_v1, edge_index_v0v1, edge_index_v1v0,
           W_rel0, b_rel0, W_root0, W_rel1, b_rel1, W_root1, W_fc, b_fc):
    y0, y1, c0, c1 = _proj_call(
        x_v0, x_v1,
        W_rel0, W_root0, b_rel0.reshape(1, H),
        W_rel1, W_root1, b_rel1.reshape(1, H))
    y0p = jnp.pad(y0, ((0, N_PAD - N), (0, 0)))
    y1p = jnp.pad(y1, ((0, N_PAD - N), (0, 0)))
    s0, d0 = _prep_edges(edge_index_v0v1)
    s1, d1 = _prep_edges(edge_index_v1v0)
    agg00, agg01, agg10, agg11 = _sc_call(y0p, y1p, s0, d0, s1, d1)
    out = _out_call(agg00, agg01, agg10, agg11, c0, c1,
                    W_fc, b_fc.reshape(1, 1))
    return out
